# baseline (speedup 1.0000x reference)
import jax
import jax.numpy as jnp
from jax.experimental import pallas as pl
from jax.experimental.pallas import tpu as pltpu

_C = 16
_R = 128
_H = _R // 2


def _half_sums(lbl, logits, d, tgt):
    ci = jax.lax.broadcasted_iota(jnp.int32, (_C, _H, _R), 0)
    mx = jnp.max(lbl, axis=0, keepdims=True)
    cls = jnp.min(jnp.where(lbl == mx, ci, _C), axis=0, keepdims=True)
    onehot = (ci == cls).astype(jnp.float32)

    x = jnp.sum(logits * onehot, axis=0)
    sp = jnp.log1p(jnp.exp(-jnp.abs(x)))
    tA = jnp.maximum(x, 0.0) - x * d + sp
    tB = tA + x * (2.0 * d - 1.0)

    oh_tgt = onehot * tgt[None]
    return (jnp.sum(onehot * tA[None], axis=(1, 2)),
            jnp.sum(onehot, axis=(1, 2)),
            jnp.sum(oh_tgt * tB[None], axis=(1, 2)),
            jnp.sum(oh_tgt, axis=(1, 2)))


def _loss_body(tsi_ref, logits_hbm, labelsT_hbm, domain_ref, outA_ref, outB_ref,
               lbl0, lbl1, log0, log1, s0, s1, s2, s3):
    cps = []
    for h, (lb, lg, sl, sg) in enumerate(((lbl0, log0, s0, s1),
                                          (lbl1, log1, s2, s3))):
        sel = pl.ds(h * _H, _H)
        c_l = pltpu.make_async_copy(labelsT_hbm.at[:, sel, :], lb, sl)
        c_g = pltpu.make_async_copy(logits_hbm.at[:, sel, :], lg, sg)
        c_l.start()
        c_g.start()
        cps.append((c_l, c_g))

    acc = None
    for h, (lb, lg, _, _) in enumerate(((lbl0, log0, s0, s1),
                                        (lbl1, log1, s2, s3))):
        cps[h][0].wait()
        cps[h][1].wait()
        sel = pl.ds(h * _H, _H)
        d = domain_ref[sel, :]
        bidx = (((h * _H) + jax.lax.broadcasted_iota(jnp.int32, (_H, _R), 0)) * _R
                + jax.lax.broadcasted_iota(jnp.int32, (_H, _R), 1))
        tgt = (bidx >= tsi_ref[0]).astype(jnp.float32)
        part = _half_sums(lb[...], lg[...], d, tgt)
        acc = part if acc is None else tuple(a + p for a, p in zip(acc, part))

    sumA, cntA, sumB, cntB = acc
    lossA = jnp.sum(sumA / cntA) * (1.0 / _C)
    lossB = jnp.sum(sumB / cntB) * (1.0 / _C)
    outA_ref[...] = jnp.broadcast_to(lossA, (1, 1))
    outB_ref[...] = jnp.broadcast_to(lossB, (1, 1))


def kernel(logits_list, labels, domain, target_start_id):
    logits3 = logits_list.reshape(_C, _R, _R)
    dom = domain.reshape(_R, _R)
    tsi = jnp.asarray(target_start_id, jnp.int32).reshape(1)

    outA, outB = pl.pallas_call(
        _loss_body,
        out_shape=(jax.ShapeDtypeStruct((1, 1), jnp.float32),
                   jax.ShapeDtypeStruct((1, 1), jnp.float32)),
        in_specs=[
            pl.BlockSpec(memory_space=pltpu.SMEM),
            pl.BlockSpec(memory_space=pl.ANY),
            pl.BlockSpec(memory_space=pl.ANY),
            pl.BlockSpec(memory_space=pltpu.VMEM),
        ],
        scratch_shapes=[pltpu.VMEM((_C, _H, _R), jnp.float32)] * 4
                       + [pltpu.SemaphoreType.DMA] * 4,
    )(tsi, logits3, labels.T.reshape(_C, _R, _R), dom)
    return (outA[0, 0], outB[0, 0])


# speedup vs baseline: 1.1069x; 1.1069x over previous
import jax
import jax.numpy as jnp
from jax.experimental import pallas as pl
from jax.experimental.pallas import tpu as pltpu

_C = 16
_R = 128


def _loss_body(tsi_ref, logits_ref, labels_ref, domain_ref, outA_ref, outB_ref):
    lbl = labels_ref[...]
    ci = jax.lax.broadcasted_iota(jnp.int32, (_C, _R, _R), 0)
    mx = jnp.max(lbl, axis=0, keepdims=True)
    cls = jnp.min(jnp.where(lbl == mx, ci, _C), axis=0, keepdims=True)
    onehot = (ci == cls).astype(jnp.float32)

    x = jnp.sum(logits_ref[...] * onehot, axis=0)
    d = domain_ref[...]
    sp = jnp.log1p(jnp.exp(-jnp.abs(x)))
    tA = jnp.maximum(x, 0.0) - x * d + sp
    tB = tA + x * (2.0 * d - 1.0)

    bidx = (jax.lax.broadcasted_iota(jnp.int32, (_R, _R), 0) * _R
            + jax.lax.broadcasted_iota(jnp.int32, (_R, _R), 1))
    tgt = (bidx >= tsi_ref[0]).astype(jnp.float32)

    sumA = jnp.sum(onehot * tA[None], axis=(1, 2))
    cntA = jnp.sum(onehot, axis=(1, 2))
    oh_tgt = onehot * tgt[None]
    sumB = jnp.sum(oh_tgt * tB[None], axis=(1, 2))
    cntB = jnp.sum(oh_tgt, axis=(1, 2))

    lossA = jnp.sum(sumA / cntA) * (1.0 / _C)
    lossB = jnp.sum(sumB / cntB) * (1.0 / _C)
    outA_ref[...] = jnp.broadcast_to(lossA, (1, 1))
    outB_ref[...] = jnp.broadcast_to(lossB, (1, 1))


def kernel(logits_list, labels, domain, target_start_id):
    logits3 = logits_list.reshape(_C, _R, _R)
    dom = domain.reshape(_R, _R)
    tsi = jnp.asarray(target_start_id, jnp.int32).reshape(1)

    outA, outB = pl.pallas_call(
        _loss_body,
        out_shape=(jax.ShapeDtypeStruct((1, 1), jnp.float32),
                   jax.ShapeDtypeStruct((1, 1), jnp.float32)),
        in_specs=[
            pl.BlockSpec(memory_space=pltpu.SMEM),
            pl.BlockSpec(memory_space=pltpu.VMEM),
            pl.BlockSpec(memory_space=pltpu.VMEM),
            pl.BlockSpec(memory_space=pltpu.VMEM),
        ],
    )(tsi, logits3, labels.T.reshape(_C, _R, _R), dom)
    return (outA[0, 0], outB[0, 0])
